# baseline (speedup 1.0000x reference)
import functools

import jax
import jax.numpy as jnp
from jax import lax
from jax.experimental import pallas as pl
from jax.experimental.pallas import tpu as pltpu
from jax.experimental.pallas import tpu_sc as plsc

NUM_BINS = 61928
EMBED_DIM = 16
BATCH = 16384
SPAN = 8

_info = plsc.get_sparse_core_info()
NC, NS, L = _info.num_cores, _info.num_subcores, _info.num_lanes
NW = NC * NS

FEAT_PER_W = BATCH // NW
ROWS_PER_W = FEAT_PER_W * SPAN
CHUNK = 128
NHALF = 2
HALF_ROWS = ROWS_PER_W // NHALF
HALF_FEAT = FEAT_PER_W // NHALF
CH_PER_SRC = HALF_ROWS // (2 * CHUNK)


def _make_kernel():
    mesh = plsc.VectorSubcoreMesh(core_axis_name="c", subcore_axis_name="s")

    @functools.partial(
        pl.kernel,
        mesh=mesh,
        out_type=jax.ShapeDtypeStruct((BATCH, EMBED_DIM), jnp.float32),
        scratch_types=[
            pltpu.VMEM((ROWS_PER_W,), jnp.int32),
            pltpu.VMEM((HALF_ROWS, EMBED_DIM), jnp.float32),
            pltpu.VMEM((FEAT_PER_W, EMBED_DIM), jnp.float32),
            pltpu.VMEM_SHARED((NUM_BINS, EMBED_DIM), jnp.float32),
            pltpu.SemaphoreType.DMA,
            pltpu.SemaphoreType.DMA,
        ],
        compiler_params=pltpu.CompilerParams(use_tc_tiling_on_sc=False),
    )
    def k(idx_hbm, table_hbm, out_hbm, idx_v, rows_v, out_v, tab_sh, sem_h, sem_s):
        wid = lax.axis_index("s") * NC + lax.axis_index("c")
        row_base = wid * ROWS_PER_W
        feat_base = wid * FEAT_PER_W

        pltpu.sync_copy(idx_hbm.at[pl.ds(row_base, ROWS_PER_W)], idx_v)

        def fire_h(half, j):
            src_off = half * HALF_ROWS + j * CHUNK
            pltpu.async_copy(
                table_hbm.at[idx_v.at[pl.ds(src_off, CHUNK)]],
                rows_v.at[pl.ds(j * CHUNK, CHUNK)],
                sem_h,
            )

        def fire_s(half, j):
            src_off = half * HALF_ROWS + HALF_ROWS // 2 + j * CHUNK
            pltpu.async_copy(
                tab_sh.at[idx_v.at[pl.ds(src_off, CHUNK)]],
                rows_v.at[pl.ds(HALF_ROWS // 2 + j * CHUNK, CHUNK)],
                sem_s,
            )

        def drain_h(half, j):
            src_off = half * HALF_ROWS + j * CHUNK
            pltpu.make_async_copy(
                table_hbm.at[idx_v.at[pl.ds(src_off, CHUNK)]],
                rows_v.at[pl.ds(j * CHUNK, CHUNK)],
                sem_h,
            ).wait()

        def drain_s(half, j):
            src_off = half * HALF_ROWS + HALF_ROWS // 2 + j * CHUNK
            pltpu.make_async_copy(
                tab_sh.at[idx_v.at[pl.ds(src_off, CHUNK)]],
                rows_v.at[pl.ds(HALF_ROWS // 2 + j * CHUNK, CHUNK)],
                sem_s,
            ).wait()

        inv = jnp.float32(1.0 / SPAN)

        def pool_half(half):
            def pool_body(f, carry):
                r = f * SPAN
                acc = rows_v[r, :]
                for s in range(1, SPAN):
                    acc = acc + rows_v[r + s, :]
                out_v[half * HALF_FEAT + f, :] = acc * inv
                return carry

            lax.fori_loop(0, HALF_FEAT, pool_body, 0, unroll=False)

        for j in range(CH_PER_SRC):
            fire_h(0, j)

        @pl.when(lax.axis_index("s") == 0)
        def _stage():
            pltpu.sync_copy(table_hbm, tab_sh)

        plsc.subcore_barrier()

        for j in range(CH_PER_SRC):
            fire_s(0, j)
        for j in range(CH_PER_SRC):
            drain_h(0, j)
        for j in range(CH_PER_SRC):
            drain_s(0, j)
        pool_half(0)

        for j in range(CH_PER_SRC):
            fire_h(1, j)
        for j in range(CH_PER_SRC):
            fire_s(1, j)
        for j in range(CH_PER_SRC):
            drain_h(1, j)
        for j in range(CH_PER_SRC):
            drain_s(1, j)
        pool_half(1)

        pltpu.sync_copy(out_v, out_hbm.at[pl.ds(feat_base, FEAT_PER_W)])

    return k


_sc_kernel = _make_kernel()


def kernel(bin_idxs, table):
    idx_flat = bin_idxs.astype(jnp.int32).reshape(BATCH * SPAN)
    return _sc_kernel(idx_flat, table)


# speedup vs baseline: 2.1419x; 2.1419x over previous
import functools

import jax
import jax.numpy as jnp
from jax import lax
from jax.experimental import pallas as pl
from jax.experimental.pallas import tpu as pltpu
from jax.experimental.pallas import tpu_sc as plsc

NUM_BINS = 61928
EMBED_DIM = 16
BATCH = 16384
SPAN = 8

_info = plsc.get_sparse_core_info()
NC, NS, L = _info.num_cores, _info.num_subcores, _info.num_lanes
NREP = NC
FEAT_PER_REP = BATCH // NREP
FCHUNK = 1024
NFCHUNK = FEAT_PER_REP // FCHUNK
GROUPS = FCHUNK // L


def _make_kernel():
    mesh = plsc.VectorSubcoreMesh(core_axis_name="c", subcore_axis_name="s")

    @functools.partial(
        pl.kernel,
        mesh=mesh,
        out_type=jax.ShapeDtypeStruct((EMBED_DIM, BATCH), jnp.float32),
        scratch_types=[
            pltpu.VMEM((NUM_BINS,), jnp.float32),
            pltpu.VMEM((2, SPAN, FCHUNK), jnp.int32),
            pltpu.VMEM((FEAT_PER_REP,), jnp.float32),
            pltpu.SemaphoreType.DMA,
            pltpu.SemaphoreType.DMA,
        ],
        compiler_params=pltpu.CompilerParams(
            use_tc_tiling_on_sc=False, needs_layout_passes=False
        ),
    )
    def k(idx_hbm, tab_hbm, out_hbm, col_v, idx_v, out_v, sem_a, sem_b):
        col_id = lax.axis_index("s")
        rep = lax.axis_index("c")
        feat_base = rep * FEAT_PER_REP
        sems = (sem_a, sem_b)

        def idx_copy(chunk, buf):
            return pltpu.make_async_copy(
                idx_hbm.at[:, pl.ds(feat_base + chunk * FCHUNK, FCHUNK)],
                idx_v.at[buf],
                sems[buf],
            )

        idx_copy(0, 0).start()
        pltpu.sync_copy(tab_hbm.at[col_id], col_v)

        inv = jnp.float32(1.0 / SPAN)

        for chunk in range(NFCHUNK):
            buf = chunk % 2
            if chunk + 1 < NFCHUNK:
                idx_copy(chunk + 1, 1 - buf).start()
            idx_copy(chunk, buf).wait()

            def group_body(g, carry, buf=buf, chunk=chunk):
                f0 = g * L
                acc = jnp.zeros((L,), jnp.float32)
                for s in range(SPAN):
                    bins = idx_v[buf, s, pl.ds(f0, L)]
                    acc = acc + plsc.load_gather(col_v, [bins])
                out_v[pl.ds(chunk * FCHUNK + f0, L)] = acc * inv
                return carry

            lax.fori_loop(0, GROUPS, group_body, 0, unroll=False)

        pltpu.sync_copy(out_v, out_hbm.at[col_id, pl.ds(feat_base, FEAT_PER_REP)])

    return k


_sc_kernel = _make_kernel()


def kernel(bin_idxs, table):
    idx_t = jnp.transpose(bin_idxs.astype(jnp.int32))
    tab_t = jnp.transpose(table)
    parts = _sc_kernel(idx_t, tab_t)
    return jnp.transpose(parts)
